# baseline (speedup 1.0000x reference)
import functools

import jax
import jax.numpy as jnp
from jax import lax
from jax.experimental import pallas as pl
from jax.experimental.pallas import tpu as pltpu
from jax.experimental.pallas import tpu_sc as plsc


_NW = 32
_SUB = 128
_NSUB = 2
_SUPER = _SUB * _NSUB


def _sc_gather(table, idx1d, row_base, nrows):
    assert nrows % _SUPER == 0 and row_base % 8 == 0
    h = table.shape[1]
    nsuper = nrows // _SUPER
    per = (nsuper + _NW - 1) // _NW
    per2 = (per + 1) // 2
    mesh = plsc.VectorSubcoreMesh(core_axis_name="c", subcore_axis_name="s")

    @functools.partial(
        pl.kernel,
        mesh=mesh,
        out_type=jax.ShapeDtypeStruct((nrows, h), table.dtype),
        scratch_types=[
            pltpu.VMEM((_SUPER,), jnp.int32),
            pltpu.VMEM((_SUPER,), jnp.int32),
            pltpu.VMEM((2 * _SUPER, h), table.dtype),
            pltpu.SemaphoreType.DMA,
            pltpu.SemaphoreType.DMA,
            pltpu.SemaphoreType.DMA,
        ],
    )
    def gk(table_hbm, idx_hbm, out_hbm, idx_v0, idx_v1, rows_v, semg, semo0, semo1):
        wid = lax.axis_index("s") * 2 + lax.axis_index("c")
        semo = [semo0, semo1]
        idxs = [idx_v0, idx_v1]

        def step(j, b, first_round):
            sc = j * _NW + wid

            @pl.when(sc < nsuper)
            def _():
                base = sc * _SUPER
                rows_b = rows_v.at[pl.ds(b * _SUPER, _SUPER)]
                out_slc = out_hbm.at[pl.ds(base, _SUPER)]
                if not first_round:
                    pltpu.make_async_copy(rows_b, out_slc, semo[b]).wait()
                pltpu.sync_copy(
                    idx_hbm.at[pl.ds(row_base + base, _SUPER)], idxs[b])
                copies = [
                    pltpu.async_copy(
                        table_hbm.at[idxs[b].at[pl.ds(i * _SUB, _SUB)]],
                        rows_b.at[pl.ds(i * _SUB, _SUB)],
                        semg,
                    )
                    for i in range(_NSUB)
                ]
                for cp in copies:
                    cp.wait()
                pltpu.async_copy(rows_b, out_slc, semo[b])

        for b in range(2):
            step(b, b, True)

        def body(jj, carry):
            for b in range(2):
                step(2 + jj * 2 + b, b, False)
            return carry

        lax.fori_loop(0, per2 - 1, body, 0)

        for b in range(2):
            cnt = (nsuper - wid + _NW - 1) // _NW

            @pl.when(cnt > b)
            def _():
                pltpu.make_async_copy(
                    rows_v.at[pl.ds(b * _SUPER, _SUPER)],
                    out_hbm.at[pl.ds(0, _SUPER)],
                    semo[b],
                ).wait()

    return gk(table, idx1d)


_NB = 400


def _gelu(x):
    return 0.5 * x * (1.0 + lax.erf(x * 0.7071067811865476))


def _bdot(x, w):
    return jnp.dot(x.astype(jnp.bfloat16), w, preferred_element_type=jnp.float32)


def _ln(x, s, o):
    mu = jnp.mean(x, axis=-1, keepdims=True)
    xc = x - mu
    var = jnp.mean(xc * xc, axis=-1, keepdims=True)
    return s * xc * lax.rsqrt(var + 1e-5) + o


def _enc_body(*refs):
    hv_ref, he_ref, c1_ref, ma_ref, mv_ref = refs[:5]
    (w1a_ref, b1_ref, w1b_ref, w1c_ref, w2_ref, b2_ref, w3_ref, b3_ref,
     win_ref, bin_ref, wout_ref, bout_ref,
     n1s_ref, n1o_ref, n2s_ref, n2o_ref) = refs[5:21]
    hvout_ref = refs[-1]

    hv = hv_ref[...]
    he = he_ref[...]
    nb, h = hv.shape
    eb = he.shape[0]
    k = eb // nb

    a1 = _bdot(hv, w1a_ref[...]) + b1_ref[...]
    x = _bdot(he, w1b_ref[...]) + _bdot(c1_ref[...], w1c_ref[...])
    x = x.reshape(nb, k, h) + a1[:, None, :]
    m = _gelu(x).reshape(eb, h)
    m = _gelu(_bdot(m, w2_ref[...]) + b2_ref[...])
    m = _bdot(m, w3_ref[...]) + b3_ref[...]
    m = m.reshape(nb, k, h) * ma_ref[...][:, :, None]
    dh = jnp.sum(m, axis=1) * (1.0 / 30.0)

    h1 = _ln(hv + dh, n1s_ref[...], n1o_ref[...])
    f = _gelu(_bdot(h1, win_ref[...]) + bin_ref[...])
    f = _bdot(f, wout_ref[...]) + bout_ref[...]
    h2 = _ln(h1 + f, n2s_ref[...], n2o_ref[...])
    hvout_ref[...] = mv_ref[...] * h2


def _edge_body(*refs):
    he_ref, c2_ref, hv_ref = refs[:3]
    (w11a_ref, b11_ref, w11b_ref, w11c_ref, w12_ref, b12_ref,
     w13_ref, b13_ref, n3s_ref, n3o_ref) = refs[3:13]
    heout_ref = refs[-1]

    he = he_ref[...]
    eb, h = he.shape
    hv = hv_ref[...]
    nb = hv.shape[0]
    k = eb // nb

    a2 = _bdot(hv, w11a_ref[...]) + b11_ref[...]
    x = _bdot(he, w11b_ref[...]) + _bdot(c2_ref[...], w11c_ref[...])
    x = x.reshape(nb, k, h) + a2[:, None, :]
    m = _gelu(x).reshape(eb, h)
    m = _gelu(_bdot(m, w12_ref[...]) + b12_ref[...])
    m = _bdot(m, w13_ref[...]) + b13_ref[...]
    heout_ref[...] = _ln(he + m, n3s_ref[...], n3o_ref[...])


def _full(shape):
    return pl.BlockSpec(shape, lambda i: (0, 0))


_ANY = pl.BlockSpec(memory_space=pl.ANY)

_PARALLEL = pltpu.CompilerParams(dimension_semantics=("parallel",))


@jax.jit
def kernel(h_V, h_E, mask_V, mask_attend,
           W1_w, W1_b, W2_w, W2_b, W3_w, W3_b,
           W11_w, W11_b, W12_w, W12_b, W13_w, W13_b,
           Win_w, Win_b, Wout_w, Wout_b,
           n1_s, n1_o, n2_s, n2_o, n3_s, n3_o, E_idx):
    b, n, h = h_V.shape
    k = E_idx.shape[-1]
    ff = Win_w.shape[1]
    nb = _NB
    eb = nb * k

    hv = h_V.reshape(n, h)
    he = h_E.reshape(n * k, h)
    eidx = E_idx.reshape(n * k)
    ma = mask_attend.reshape(n, k)
    mv = mask_V.reshape(n, 1)

    wb = lambda v: v.astype(jnp.bfloat16)
    w1a, w1b, w1c = wb(W1_w[:h]), wb(W1_w[h:2 * h]), wb(W1_w[2 * h:])
    w11a, w11b, w11c = wb(W11_w[:h]), wb(W11_w[h:2 * h]), wb(W11_w[2 * h:])
    w2, w3, win, wout = wb(W2_w), wb(W3_w), wb(Win_w), wb(Wout_w)
    w12, w13 = wb(W12_w), wb(W13_w)
    r = lambda v: v.reshape(1, -1)

    segs = [(0, 9), (9, 8), (17, 8)]

    c1s = [_sc_gather(hv, eidx, off * eb, steps * eb) for off, steps in segs]

    enc_w = [w1a, r(W1_b), w1b, w1c, w2, r(W2_b), w3, r(W3_b),
             win, r(Win_b), wout, r(Wout_b),
             r(n1_s), r(n1_o), r(n2_s), r(n2_o)]
    enc_w_specs = [_full(x.shape) for x in enc_w]
    hv_full = None
    for (off, steps), c1_s in zip(segs, c1s):
        g = lambda i, o=off: (i + o, 0)
        l = lambda i: (i, 0)
        extra_in = [] if hv_full is None else [hv_full]
        hv_full = pl.pallas_call(
            _enc_body,
            grid=(steps,),
            in_specs=[
                pl.BlockSpec((nb, h), g),
                pl.BlockSpec((eb, h), g),
                pl.BlockSpec((eb, h), l),
                pl.BlockSpec((nb, k), g),
                pl.BlockSpec((nb, 1), g),
            ] + enc_w_specs + [_ANY] * len(extra_in),
            out_specs=pl.BlockSpec((nb, h), g),
            out_shape=jax.ShapeDtypeStruct((n, h), jnp.float32),
            input_output_aliases={21: 0} if extra_in else {},
            compiler_params=_PARALLEL,
        )(hv, he, c1_s, ma, mv, *enc_w, *extra_in)

    c2s = [_sc_gather(hv_full, eidx, off * eb, steps * eb) for off, steps in segs]

    edge_w = [w11a, r(W11_b), w11b, w11c, w12, r(W12_b), w13, r(W13_b),
              r(n3_s), r(n3_o)]
    edge_w_specs = [_full(x.shape) for x in edge_w]
    he_full = None
    for (off, steps), c2_s in zip(segs, c2s):
        g = lambda i, o=off: (i + o, 0)
        l = lambda i: (i, 0)
        extra_in = [] if he_full is None else [he_full]
        he_full = pl.pallas_call(
            _edge_body,
            grid=(steps,),
            in_specs=[
                pl.BlockSpec((eb, h), g),
                pl.BlockSpec((eb, h), l),
                pl.BlockSpec((nb, h), g),
            ] + edge_w_specs + [_ANY] * len(extra_in),
            out_specs=pl.BlockSpec((eb, h), g),
            out_shape=jax.ShapeDtypeStruct((n * k, h), jnp.float32),
            input_output_aliases={13: 0} if extra_in else {},
            compiler_params=_PARALLEL,
        )(he, c2_s, hv_full, *edge_w, *extra_in)

    return hv_full.reshape(b, n, h), he_full.reshape(b, n, k, h)


# speedup vs baseline: 1.0425x; 1.0425x over previous
import functools

import jax
import jax.numpy as jnp
from jax import lax
from jax.experimental import pallas as pl
from jax.experimental.pallas import tpu as pltpu
from jax.experimental.pallas import tpu_sc as plsc


_NW = 32
_SUB = 128
_NSUB = 5
_SUPER = _SUB * _NSUB


def _sc_gather(table, idx1d, row_base, nrows):
    assert nrows % _SUPER == 0 and row_base % 8 == 0
    h = table.shape[1]
    nsuper = nrows // _SUPER
    per = (nsuper + _NW - 1) // _NW
    mesh = plsc.VectorSubcoreMesh(core_axis_name="c", subcore_axis_name="s")

    @functools.partial(
        pl.kernel,
        mesh=mesh,
        out_type=jax.ShapeDtypeStruct((nrows, h), table.dtype),
        scratch_types=[
            pltpu.VMEM((_SUPER,), jnp.int32),
            pltpu.VMEM((_SUPER, h), table.dtype),
            pltpu.SemaphoreType.DMA,
        ],
    )
    def gk(table_hbm, idx_hbm, out_hbm, idx_v, rows_v, sem):
        wid = lax.axis_index("s") * 2 + lax.axis_index("c")

        def body(j, carry):
            sc = j * _NW + wid

            @pl.when(sc < nsuper)
            def _():
                base = sc * _SUPER
                pltpu.sync_copy(idx_hbm.at[pl.ds(row_base + base, _SUPER)], idx_v)
                copies = [
                    pltpu.async_copy(
                        table_hbm.at[idx_v.at[pl.ds(i * _SUB, _SUB)]],
                        rows_v.at[pl.ds(i * _SUB, _SUB)],
                        sem,
                    )
                    for i in range(_NSUB)
                ]
                for cp in copies:
                    cp.wait()
                pltpu.sync_copy(rows_v, out_hbm.at[pl.ds(base, _SUPER)])

            return carry

        lax.fori_loop(0, per, body, 0)

    return gk(table, idx1d)


_NB = 400


def _gelu(x):
    return 0.5 * x * (1.0 + lax.erf(x * 0.7071067811865476))


def _bdot(x, w):
    return jnp.dot(x.astype(jnp.bfloat16), w, preferred_element_type=jnp.float32)


def _ln(x, s, o):
    mu = jnp.mean(x, axis=-1, keepdims=True)
    xc = x - mu
    var = jnp.mean(xc * xc, axis=-1, keepdims=True)
    return s * xc * lax.rsqrt(var + 1e-5) + o


def _enc_body(*refs):
    hv_ref, he_ref, c1_ref, ma_ref, mv_ref = refs[:5]
    (w1a_ref, b1_ref, w1b_ref, w1c_ref, w2_ref, b2_ref, w3_ref, b3_ref,
     win_ref, bin_ref, wout_ref, bout_ref,
     n1s_ref, n1o_ref, n2s_ref, n2o_ref) = refs[5:21]
    hvout_ref = refs[-1]

    hv = hv_ref[...]
    he = he_ref[...]
    nb, h = hv.shape
    eb = he.shape[0]
    k = eb // nb

    a1 = _bdot(hv, w1a_ref[...]) + b1_ref[...]
    x = _bdot(he, w1b_ref[...]) + _bdot(c1_ref[...], w1c_ref[...])
    x = x.reshape(nb, k, h) + a1[:, None, :]
    m = _gelu(x).reshape(eb, h)
    m = _gelu(_bdot(m, w2_ref[...]) + b2_ref[...])
    m = _bdot(m, w3_ref[...]) + b3_ref[...]
    m = m.reshape(nb, k, h) * ma_ref[...][:, :, None]
    dh = jnp.sum(m, axis=1) * (1.0 / 30.0)

    h1 = _ln(hv + dh, n1s_ref[...], n1o_ref[...])
    f = _gelu(_bdot(h1, win_ref[...]) + bin_ref[...])
    f = _bdot(f, wout_ref[...]) + bout_ref[...]
    h2 = _ln(h1 + f, n2s_ref[...], n2o_ref[...])
    hvout_ref[...] = mv_ref[...] * h2


def _edge_body(*refs):
    he_ref, c2_ref, hv_ref = refs[:3]
    (w11a_ref, b11_ref, w11b_ref, w11c_ref, w12_ref, b12_ref,
     w13_ref, b13_ref, n3s_ref, n3o_ref) = refs[3:13]
    heout_ref = refs[-1]

    he = he_ref[...]
    eb, h = he.shape
    hv = hv_ref[...]
    nb = hv.shape[0]
    k = eb // nb

    a2 = _bdot(hv, w11a_ref[...]) + b11_ref[...]
    x = _bdot(he, w11b_ref[...]) + _bdot(c2_ref[...], w11c_ref[...])
    x = x.reshape(nb, k, h) + a2[:, None, :]
    m = _gelu(x).reshape(eb, h)
    m = _gelu(_bdot(m, w12_ref[...]) + b12_ref[...])
    m = _bdot(m, w13_ref[...]) + b13_ref[...]
    heout_ref[...] = _ln(he + m, n3s_ref[...], n3o_ref[...])


def _full(shape):
    return pl.BlockSpec(shape, lambda i: (0, 0))


_ANY = pl.BlockSpec(memory_space=pl.ANY)

_PARALLEL = pltpu.CompilerParams(dimension_semantics=("parallel",))


@jax.jit
def kernel(h_V, h_E, mask_V, mask_attend,
           W1_w, W1_b, W2_w, W2_b, W3_w, W3_b,
           W11_w, W11_b, W12_w, W12_b, W13_w, W13_b,
           Win_w, Win_b, Wout_w, Wout_b,
           n1_s, n1_o, n2_s, n2_o, n3_s, n3_o, E_idx):
    b, n, h = h_V.shape
    k = E_idx.shape[-1]
    ff = Win_w.shape[1]
    nb = _NB
    eb = nb * k

    hv = h_V.reshape(n, h)
    he = h_E.reshape(n * k, h)
    eidx = E_idx.reshape(n * k)
    ma = mask_attend.reshape(n, k)
    mv = mask_V.reshape(n, 1)

    wb = lambda v: v.astype(jnp.bfloat16)
    w1a, w1b, w1c = wb(W1_w[:h]), wb(W1_w[h:2 * h]), wb(W1_w[2 * h:])
    w11a, w11b, w11c = wb(W11_w[:h]), wb(W11_w[h:2 * h]), wb(W11_w[2 * h:])
    w2, w3, win, wout = wb(W2_w), wb(W3_w), wb(Win_w), wb(Wout_w)
    w12, w13 = wb(W12_w), wb(W13_w)
    r = lambda v: v.reshape(1, -1)

    segs = [(0, 9), (9, 8), (17, 8)]

    c1s = [_sc_gather(hv, eidx, off * eb, steps * eb) for off, steps in segs]

    enc_w = [w1a, r(W1_b), w1b, w1c, w2, r(W2_b), w3, r(W3_b),
             win, r(Win_b), wout, r(Wout_b),
             r(n1_s), r(n1_o), r(n2_s), r(n2_o)]
    enc_w_specs = [_full(x.shape) for x in enc_w]
    hv_full = None
    for (off, steps), c1_s in zip(segs, c1s):
        g = lambda i, o=off: (i + o, 0)
        l = lambda i: (i, 0)
        extra_in = [] if hv_full is None else [hv_full]
        hv_full = pl.pallas_call(
            _enc_body,
            grid=(steps,),
            in_specs=[
                pl.BlockSpec((nb, h), g),
                pl.BlockSpec((eb, h), g),
                pl.BlockSpec((eb, h), l),
                pl.BlockSpec((nb, k), g),
                pl.BlockSpec((nb, 1), g),
            ] + enc_w_specs + [_ANY] * len(extra_in),
            out_specs=pl.BlockSpec((nb, h), g),
            out_shape=jax.ShapeDtypeStruct((n, h), jnp.float32),
            input_output_aliases={21: 0} if extra_in else {},
            compiler_params=_PARALLEL,
        )(hv, he, c1_s, ma, mv, *enc_w, *extra_in)

    c2s = [_sc_gather(hv_full, eidx, off * eb, steps * eb) for off, steps in segs]

    edge_w = [w11a, r(W11_b), w11b, w11c, w12, r(W12_b), w13, r(W13_b),
              r(n3_s), r(n3_o)]
    edge_w_specs = [_full(x.shape) for x in edge_w]
    he_full = None
    for (off, steps), c2_s in zip(segs, c2s):
        g = lambda i, o=off: (i + o, 0)
        l = lambda i: (i, 0)
        extra_in = [] if he_full is None else [he_full]
        he_full = pl.pallas_call(
            _edge_body,
            grid=(steps,),
            in_specs=[
                pl.BlockSpec((eb, h), g),
                pl.BlockSpec((eb, h), l),
                pl.BlockSpec((nb, h), g),
            ] + edge_w_specs + [_ANY] * len(extra_in),
            out_specs=pl.BlockSpec((eb, h), g),
            out_shape=jax.ShapeDtypeStruct((n * k, h), jnp.float32),
            input_output_aliases={13: 0} if extra_in else {},
            compiler_params=_PARALLEL,
        )(he, c2_s, hv_full, *edge_w, *extra_in)

    return hv_full.reshape(b, n, h), he_full.reshape(b, n, k, h)
